# baseline (speedup 1.0000x reference)
import functools
import math

import jax
import jax.numpy as jnp
from jax import lax
from jax.experimental import pallas as pl
from jax.experimental.pallas import tpu as pltpu
from jax.experimental.pallas import tpu_sc as plsc

VOCAB = 32768
SMOOTH = 0.1
CONF = 1.0 - SMOOTH
EPS = SMOOTH / (VOCAB - 2)
K_CONST = (VOCAB - 2) * EPS * math.log(EPS) + CONF * math.log(CONF)

N_ROWS = 4096
BLK_R = 256
BLK_V = 4096

SC_NC = 2
SC_NS = 16
SC_NW = SC_NC * SC_NS
SC_CHUNK = N_ROWS // SC_NW


def _sc_gather_body(xflat_hbm, tgt_hbm, out_hbm, tgt_v, idx_v, val_v, acc_v,
                    sem):
    wid = lax.axis_index("s") * SC_NC + lax.axis_index("c")
    base = wid * SC_CHUNK
    pltpu.sync_copy(tgt_hbm.at[pl.ds(base, SC_CHUNK)], tgt_v)
    for k in range(SC_CHUNK // 16):
        t16 = tgt_v[pl.ds(k * 16, 16)]
        row = base + k * 16 + lax.iota(jnp.int32, 16)
        idx_v[pl.ds(k * 16, 16)] = row * VOCAB + t16
    pltpu.async_copy(xflat_hbm.at[idx_v], val_v, sem).wait()
    acc = jnp.zeros((16,), jnp.float32)
    for k in range(SC_CHUNK // 16):
        t16 = tgt_v[pl.ds(k * 16, 16)]
        v16 = val_v[pl.ds(k * 16, 16)]
        acc = acc + jnp.where(t16 != 0, v16, 0.0)
    acc_v[...] = acc
    pltpu.sync_copy(acc_v, out_hbm.at[wid])


_sc_gather = functools.partial(
    pl.kernel,
    out_type=jax.ShapeDtypeStruct((SC_NW, 16), jnp.float32),
    mesh=plsc.VectorSubcoreMesh(core_axis_name="c", subcore_axis_name="s"),
    scratch_types=[
        pltpu.VMEM((SC_CHUNK,), jnp.int32),
        pltpu.VMEM((SC_CHUNK,), jnp.int32),
        pltpu.VMEM((SC_CHUNK,), jnp.float32),
        pltpu.VMEM((16,), jnp.float32),
        pltpu.SemaphoreType.DMA,
    ],
)(_sc_gather_body)


def _tc_body(t_ref, x_ref, st_ref, out_ref, acc_ref):
    i = pl.program_id(0)
    j = pl.program_id(1)
    ni = pl.num_programs(0)
    nj = pl.num_programs(1)

    @pl.when((i == 0) & (j == 0))
    def _init():
        acc_ref[0] = 0.0
        acc_ref[1] = 0.0
        acc_ref[2] = 0.0

    t = t_ref[...]
    w = (t != 0).astype(jnp.float32)
    xs = x_ref[...]
    rs = jnp.sum(xs, axis=1, keepdims=True)
    acc_ref[0] += jnp.sum(rs * w)

    @pl.when(j == 0)
    def _col0():
        acc_ref[1] += jnp.sum(xs[:, 0:1] * w)
        acc_ref[2] += jnp.sum(w)

    @pl.when((i == ni - 1) & (j == nj - 1))
    def _fin():
        total = acc_ref[0]
        s0 = acc_ref[1]
        cnt = acc_ref[2]
        st_sum = jnp.sum(st_ref[...])
        out_ref[0] = (cnt * K_CONST - EPS * (total - s0)
                      - (CONF - EPS) * st_sum)


@jax.jit
def _loss(x, t32):
    st_parts = _sc_gather(x.reshape(-1), t32)
    n = x.shape[0]
    grid = (n // BLK_R, VOCAB // BLK_V)
    res = pl.pallas_call(
        _tc_body,
        grid=grid,
        in_specs=[
            pl.BlockSpec((BLK_R, 1), lambda i, j: (i, 0)),
            pl.BlockSpec((BLK_R, BLK_V), lambda i, j: (i, j)),
            pl.BlockSpec((SC_NW, 16), lambda i, j: (0, 0)),
        ],
        out_specs=pl.BlockSpec(memory_space=pltpu.SMEM),
        out_shape=jax.ShapeDtypeStruct((1,), jnp.float32),
        scratch_shapes=[pltpu.SMEM((3,), jnp.float32)],
    )(t32.reshape(-1, 1), x, st_parts)
    return res[0]


def kernel(x, target):
    return _loss(x, target.astype(jnp.int32))


# speedup vs baseline: 2.8351x; 2.8351x over previous
import functools
import math

import jax
import jax.numpy as jnp
from jax import lax
from jax.experimental import pallas as pl
from jax.experimental.pallas import tpu as pltpu
from jax.experimental.pallas import tpu_sc as plsc

VOCAB = 32768
SMOOTH = 0.1
CONF = 1.0 - SMOOTH
EPS = SMOOTH / (VOCAB - 2)
K_CONST = (VOCAB - 2) * EPS * math.log(EPS) + CONF * math.log(CONF)

N_ROWS = 4096
BLK_R = 256
BLK_V = 4096

SC_NC = 2
SC_NS = 16
SC_NW = SC_NC * SC_NS
SC_CHUNK = N_ROWS // SC_NW


def _sc_gather_body(xflat_hbm, tgt_hbm, out_hbm, tgt_v, idx_v, val_v, acc_v,
                    sem):
    wid = lax.axis_index("s") * SC_NC + lax.axis_index("c")
    base = wid * SC_CHUNK
    pltpu.sync_copy(tgt_hbm.at[pl.ds(base, SC_CHUNK)], tgt_v)
    for k in range(SC_CHUNK // 16):
        t16 = tgt_v[pl.ds(k * 16, 16)]
        row = base + k * 16 + lax.iota(jnp.int32, 16)
        idx_v[pl.ds(k * 16, 16)] = (
            (row >> 3) * (VOCAB * 8)
            + (t16 >> 7) * 1024
            + (row & 7) * 128
            + (t16 & 127)
        )
    pltpu.async_copy(xflat_hbm.at[idx_v], val_v, sem).wait()
    acc = jnp.zeros((16,), jnp.float32)
    for k in range(SC_CHUNK // 16):
        t16 = tgt_v[pl.ds(k * 16, 16)]
        v16 = val_v[pl.ds(k * 16, 16)]
        acc = acc + jnp.where(t16 != 0, v16, 0.0)
    acc_v[...] = acc
    pltpu.sync_copy(acc_v, out_hbm.at[wid])


_sc_gather = functools.partial(
    pl.kernel,
    out_type=jax.ShapeDtypeStruct((SC_NW, 16), jnp.float32),
    mesh=plsc.VectorSubcoreMesh(core_axis_name="c", subcore_axis_name="s"),
    scratch_types=[
        pltpu.VMEM((SC_CHUNK,), jnp.int32),
        pltpu.VMEM((SC_CHUNK,), jnp.int32),
        pltpu.VMEM((SC_CHUNK,), jnp.float32),
        pltpu.VMEM((16,), jnp.float32),
        pltpu.SemaphoreType.DMA,
    ],
)(_sc_gather_body)


def _tc_body(t_ref, x_ref, st_ref, out_ref, acc_ref):
    i = pl.program_id(0)
    j = pl.program_id(1)
    ni = pl.num_programs(0)
    nj = pl.num_programs(1)

    @pl.when((i == 0) & (j == 0))
    def _init():
        acc_ref[0] = 0.0
        acc_ref[1] = 0.0
        acc_ref[2] = 0.0

    t = t_ref[...]
    w = (t != 0).astype(jnp.float32)
    xs = x_ref[...]
    rs = jnp.sum(xs, axis=1, keepdims=True)
    acc_ref[0] += jnp.sum(rs * w)

    @pl.when(j == 0)
    def _col0():
        acc_ref[1] += jnp.sum(xs[:, 0:1] * w)
        acc_ref[2] += jnp.sum(w)

    @pl.when((i == ni - 1) & (j == nj - 1))
    def _fin():
        total = acc_ref[0]
        s0 = acc_ref[1]
        cnt = acc_ref[2]
        st_sum = jnp.sum(st_ref[...])
        out_ref[0] = (cnt * K_CONST - EPS * (total - s0)
                      - (CONF - EPS) * st_sum)


@jax.jit
def _loss(x, t32):
    n = x.shape[0]
    x_lin = (x.reshape(n // 8, 8, VOCAB // 128, 128)
             .transpose(0, 2, 1, 3).reshape(-1))
    st_parts = _sc_gather(x_lin, t32)
    n = x.shape[0]
    grid = (n // BLK_R, VOCAB // BLK_V)

    res = pl.pallas_call(
        _tc_body,
        grid=grid,
        in_specs=[
            pl.BlockSpec((BLK_R, 1), lambda i, j: (i, 0)),
            pl.BlockSpec((BLK_R, BLK_V), lambda i, j: (i, j)),
            pl.BlockSpec((SC_NW, 16), lambda i, j: (0, 0)),
        ],
        out_specs=pl.BlockSpec(memory_space=pltpu.SMEM),
        out_shape=jax.ShapeDtypeStruct((1,), jnp.float32),
        scratch_shapes=[pltpu.SMEM((3,), jnp.float32)],
    )(t32.reshape(-1, 1), x, st_parts)
    return res[0]


def kernel(x, target):
    return _loss(x, target.astype(jnp.int32))


# speedup vs baseline: 3.2042x; 1.1302x over previous
import functools
import math

import jax
import jax.numpy as jnp
from jax import lax
from jax.experimental import pallas as pl
from jax.experimental.pallas import tpu as pltpu
from jax.experimental.pallas import tpu_sc as plsc

VOCAB = 32768
SMOOTH = 0.1
CONF = 1.0 - SMOOTH
EPS = SMOOTH / (VOCAB - 2)
K_CONST = (VOCAB - 2) * EPS * math.log(EPS) + CONF * math.log(CONF)

N_ROWS = 4096
BLK_R = 256
BLK_V = 8192

SC_NC = 2
SC_NS = 16
SC_NW = SC_NC * SC_NS
SC_CHUNK = N_ROWS // SC_NW


def _sc_gather_body(xflat_hbm, tgt_hbm, out_hbm, tgt_v, idx_v, val_v, acc_v,
                    sem):
    wid = lax.axis_index("s") * SC_NC + lax.axis_index("c")
    base = wid * SC_CHUNK
    pltpu.sync_copy(tgt_hbm.at[pl.ds(base, SC_CHUNK)], tgt_v)
    for k in range(SC_CHUNK // 16):
        t16 = tgt_v[pl.ds(k * 16, 16)]
        row = base + k * 16 + lax.iota(jnp.int32, 16)
        idx_v[pl.ds(k * 16, 16)] = (
            (row >> 3) * (VOCAB * 8)
            + (t16 >> 7) * 1024
            + (row & 7) * 128
            + (t16 & 127)
        )
    pltpu.async_copy(xflat_hbm.at[idx_v], val_v, sem).wait()
    acc = jnp.zeros((16,), jnp.float32)
    for k in range(SC_CHUNK // 16):
        t16 = tgt_v[pl.ds(k * 16, 16)]
        v16 = val_v[pl.ds(k * 16, 16)]
        acc = acc + jnp.where(t16 != 0, v16, 0.0)
    acc_v[...] = acc
    pltpu.sync_copy(acc_v, out_hbm.at[wid])


_sc_gather = functools.partial(
    pl.kernel,
    out_type=jax.ShapeDtypeStruct((SC_NW, 16), jnp.float32),
    mesh=plsc.VectorSubcoreMesh(core_axis_name="c", subcore_axis_name="s"),
    scratch_types=[
        pltpu.VMEM((SC_CHUNK,), jnp.int32),
        pltpu.VMEM((SC_CHUNK,), jnp.int32),
        pltpu.VMEM((SC_CHUNK,), jnp.float32),
        pltpu.VMEM((16,), jnp.float32),
        pltpu.SemaphoreType.DMA,
    ],
)(_sc_gather_body)


def _tc_body(t_ref, x_ref, st_ref, out_ref, acc_ref):
    i = pl.program_id(0)
    j = pl.program_id(1)
    ni = pl.num_programs(0)
    nj = pl.num_programs(1)

    @pl.when((i == 0) & (j == 0))
    def _init():
        acc_ref[0] = 0.0
        acc_ref[1] = 0.0
        acc_ref[2] = 0.0

    t = t_ref[...]
    w = (t != 0).astype(jnp.float32)
    xs = x_ref[...]
    rs = jnp.sum(xs, axis=1, keepdims=True)
    acc_ref[0] += jnp.sum(rs * w)

    @pl.when(j == 0)
    def _col0():
        acc_ref[1] += jnp.sum(xs[:, 0:1] * w)
        acc_ref[2] += jnp.sum(w)

    @pl.when((i == ni - 1) & (j == nj - 1))
    def _fin():
        total = acc_ref[0]
        s0 = acc_ref[1]
        cnt = acc_ref[2]
        st_sum = jnp.sum(st_ref[...])
        out_ref[0] = (cnt * K_CONST - EPS * (total - s0)
                      - (CONF - EPS) * st_sum)


@jax.jit
def _loss(x, t32):
    n = x.shape[0]
    x_lin = (x.reshape(n // 8, 8, VOCAB // 128, 128)
             .transpose(0, 2, 1, 3).reshape(-1))
    st_parts = _sc_gather(x_lin, t32)
    n = x.shape[0]
    grid = (n // BLK_R, VOCAB // BLK_V)

    res = pl.pallas_call(
        _tc_body,
        grid=grid,
        in_specs=[
            pl.BlockSpec((BLK_R, 1), lambda i, j: (i, 0)),
            pl.BlockSpec((BLK_R, BLK_V), lambda i, j: (i, j)),
            pl.BlockSpec((SC_NW, 16), lambda i, j: (0, 0)),
        ],
        out_specs=pl.BlockSpec(memory_space=pltpu.SMEM),
        out_shape=jax.ShapeDtypeStruct((1,), jnp.float32),
        scratch_shapes=[pltpu.SMEM((3,), jnp.float32)],
    )(t32.reshape(-1, 1), x, st_parts)
    return res[0]


def kernel(x, target):
    return _loss(x, target.astype(jnp.int32))


# speedup vs baseline: 3.2078x; 1.0011x over previous
import functools
import math

import jax
import jax.numpy as jnp
from jax import lax
from jax.experimental import pallas as pl
from jax.experimental.pallas import tpu as pltpu
from jax.experimental.pallas import tpu_sc as plsc

VOCAB = 32768
SMOOTH = 0.1
CONF = 1.0 - SMOOTH
EPS = SMOOTH / (VOCAB - 2)
K_CONST = (VOCAB - 2) * EPS * math.log(EPS) + CONF * math.log(CONF)

N_ROWS = 4096
BLK_R = 128
BLK_V = 32768

SC_NC = 2
SC_NS = 16
SC_NW = SC_NC * SC_NS
SC_CHUNK = N_ROWS // SC_NW


def _sc_gather_body(xflat_hbm, tgt_hbm, out_hbm, tgt_v, idx_v, val_v, acc_v,
                    sem):
    wid = lax.axis_index("s") * SC_NC + lax.axis_index("c")
    base = wid * SC_CHUNK
    pltpu.sync_copy(tgt_hbm.at[pl.ds(base, SC_CHUNK)], tgt_v)
    for k in range(SC_CHUNK // 16):
        t16 = tgt_v[pl.ds(k * 16, 16)]
        row = base + k * 16 + lax.iota(jnp.int32, 16)
        idx_v[pl.ds(k * 16, 16)] = (
            (row >> 3) * (VOCAB * 8)
            + (t16 >> 7) * 1024
            + (row & 7) * 128
            + (t16 & 127)
        )
    pltpu.async_copy(xflat_hbm.at[idx_v], val_v, sem).wait()
    acc = jnp.zeros((16,), jnp.float32)
    for k in range(SC_CHUNK // 16):
        t16 = tgt_v[pl.ds(k * 16, 16)]
        v16 = val_v[pl.ds(k * 16, 16)]
        acc = acc + jnp.where(t16 != 0, v16, 0.0)
    acc_v[...] = acc
    pltpu.sync_copy(acc_v, out_hbm.at[wid])


_sc_gather = functools.partial(
    pl.kernel,
    out_type=jax.ShapeDtypeStruct((SC_NW, 16), jnp.float32),
    mesh=plsc.VectorSubcoreMesh(core_axis_name="c", subcore_axis_name="s"),
    scratch_types=[
        pltpu.VMEM((SC_CHUNK,), jnp.int32),
        pltpu.VMEM((SC_CHUNK,), jnp.int32),
        pltpu.VMEM((SC_CHUNK,), jnp.float32),
        pltpu.VMEM((16,), jnp.float32),
        pltpu.SemaphoreType.DMA,
    ],
)(_sc_gather_body)


def _tc_body(t_ref, x_ref, st_ref, out_ref, acc_ref):
    i = pl.program_id(0)
    j = pl.program_id(1)
    ni = pl.num_programs(0)
    nj = pl.num_programs(1)

    @pl.when((i == 0) & (j == 0))
    def _init():
        acc_ref[0] = 0.0
        acc_ref[1] = 0.0
        acc_ref[2] = 0.0

    t = t_ref[...]
    w = (t != 0).astype(jnp.float32)
    xs = x_ref[...]
    rs = jnp.sum(xs, axis=1, keepdims=True)
    acc_ref[0] += jnp.sum(rs * w)

    @pl.when(j == 0)
    def _col0():
        acc_ref[1] += jnp.sum(xs[:, 0:1] * w)
        acc_ref[2] += jnp.sum(w)

    @pl.when((i == ni - 1) & (j == nj - 1))
    def _fin():
        total = acc_ref[0]
        s0 = acc_ref[1]
        cnt = acc_ref[2]
        st_sum = jnp.sum(st_ref[...])
        out_ref[0] = (cnt * K_CONST - EPS * (total - s0)
                      - (CONF - EPS) * st_sum)


@jax.jit
def _loss(x, t32):
    n = x.shape[0]
    x_lin = (x.reshape(n // 8, 8, VOCAB // 128, 128)
             .transpose(0, 2, 1, 3).reshape(-1))
    st_parts = _sc_gather(x_lin, t32)
    n = x.shape[0]
    grid = (n // BLK_R, VOCAB // BLK_V)

    res = pl.pallas_call(
        _tc_body,
        grid=grid,
        in_specs=[
            pl.BlockSpec((BLK_R, 1), lambda i, j: (i, 0)),
            pl.BlockSpec((BLK_R, BLK_V), lambda i, j: (i, j)),
            pl.BlockSpec((SC_NW, 16), lambda i, j: (0, 0)),
        ],
        out_specs=pl.BlockSpec(memory_space=pltpu.SMEM),
        out_shape=jax.ShapeDtypeStruct((1,), jnp.float32),
        scratch_shapes=[pltpu.SMEM((3,), jnp.float32)],
    )(t32.reshape(-1, 1), x, st_parts)
    return res[0]


def kernel(x, target):
    return _loss(x, target.astype(jnp.int32))
